# baseline (speedup 1.0000x reference)
import functools

import jax
import jax.numpy as jnp
from jax import lax
from jax.experimental import pallas as pl
from jax.experimental.pallas import tpu as pltpu
from jax.experimental.pallas import tpu_sc as plsc

NC = 2
NS = 16
CH = 128
IB = 8
FG = 128
DW = 16
BN = 1000


def _mesh():
    return plsc.VectorSubcoreMesh(core_axis_name="c", subcore_axis_name="s",
                                  num_cores=NC, num_subcores=NS)


def _bcast(ref, i):
    return plsc.load_gather(ref, [jnp.full((16,), i, jnp.int32)])


TRASH_PER_TILE = 8
NBUF = 2


def _make_deg_kernel(n_nodes, ne_pad):
    nchunks = ne_pad // (NC * NS * CH)
    racc = n_nodes + NC * NS * TRASH_PER_TILE

    @functools.partial(
        pl.kernel,
        out_type=jax.ShapeDtypeStruct((NC, n_nodes, DW), jnp.float32),
        mesh=_mesh(),
        compiler_params=pltpu.CompilerParams(needs_layout_passes=False),
        scratch_types=[
            pltpu.VMEM((CH,), jnp.int32),
            pltpu.VMEM((CH, DW), jnp.float32),
            pltpu.VMEM_SHARED((racc, DW), jnp.float32),
        ],
    )
    def k(colp_hbm, ones_hbm, zeros_hbm, out_hbm, dst_v, ones_v, acc):
        cid = lax.axis_index("c")
        sid = lax.axis_index("s")
        wid = sid * NC + cid
        pltpu.sync_copy(ones_hbm, ones_v)

        @pl.when(sid == 0)
        def _():
            pltpu.sync_copy(zeros_hbm, acc)

        plsc.subcore_barrier()

        def chunk(kk, carry):
            e0 = (wid * nchunks + kk) * CH
            pltpu.sync_copy(colp_hbm.at[pl.ds(e0, CH)], dst_v)
            pltpu.sync_copy(ones_v, acc.at[dst_v], add=True)
            return carry

        lax.fori_loop(0, nchunks, chunk, 0)
        plsc.subcore_barrier()

        @pl.when(sid == 0)
        def _():
            pltpu.sync_copy(acc.at[pl.ds(0, n_nodes)], out_hbm.at[cid])

        plsc.subcore_barrier()

    return k


def _make_layer_kernel(n_nodes, ne_pad, n_groups, n_steps):
    nchunks = ne_pad // (NC * NS * CH)
    racc = n_nodes + NC * NS * TRASH_PER_TILE
    gpc = n_groups // NC
    n_blocks = n_nodes // 16
    nbt = -(-n_blocks // NS)

    @functools.partial(
        pl.kernel,
        out_type=jax.ShapeDtypeStruct((n_steps, n_groups, n_nodes, FG),
                                      jnp.float32),
        mesh=_mesh(),
        compiler_params=pltpu.CompilerParams(needs_layout_passes=False),
        scratch_types=[
            [pltpu.VMEM((CH,), jnp.int32) for _ in range(NBUF)],
            [pltpu.VMEM((CH,), jnp.int32) for _ in range(NBUF)],
            [pltpu.VMEM((CH, FG), jnp.float32) for _ in range(NBUF)],
            [pltpu.SemaphoreType.DMA for _ in range(NBUF)],
            [pltpu.SemaphoreType.DMA for _ in range(NBUF)],
            pltpu.VMEM((16, FG), jnp.float32),
            pltpu.VMEM((16, FG), jnp.float32),
            pltpu.VMEM((n_nodes,), jnp.float32),
            pltpu.VMEM_SHARED((racc, FG), jnp.float32),
        ],
    )
    def k(u0_hbm, rowp_hbm, colp_hbm, dinv2_hbm, zeros_hbm, us_hbm,
          idx_v, dst_v, rows_v, gsem, ssem, dbuf, obuf, dinv2_v, acc):
        cid = lax.axis_index("c")
        sid = lax.axis_index("s")
        wid = sid * NC + cid
        pltpu.sync_copy(dinv2_hbm, dinv2_v)
        dummy = u0_hbm.at[0, pl.ds(0, CH)]

        for step in range(n_steps):
            for gg in range(gpc):
                g = cid * gpc + gg

                @pl.when(sid == 0)
                def _():
                    pltpu.sync_copy(zeros_hbm, acc)

                plsc.subcore_barrier()

                if step == 0:
                    gsrc = u0_hbm.at[g]
                else:
                    gsrc = us_hbm.at[step - 1, g]

                def outer(q, carry, gsrc=gsrc):
                    gds = []
                    for b in range(NBUF):
                        kk = q * NBUF + b
                        e0 = (wid * nchunks + kk) * CH

                        @pl.when(q > 0)
                        def _(b=b):
                            pltpu.make_async_copy(
                                dummy, rows_v[b], ssem[b]).wait()

                        pltpu.sync_copy(rowp_hbm.at[pl.ds(e0, CH)],
                                        idx_v[b])
                        pltpu.sync_copy(colp_hbm.at[pl.ds(e0, CH)],
                                        dst_v[b])
                        gds.append(pltpu.async_copy(
                            gsrc.at[idx_v[b]], rows_v[b], gsem[b]))
                    for b in range(NBUF):
                        gds[b].wait()
                        pltpu.async_copy(rows_v[b], acc.at[dst_v[b]],
                                        ssem[b], add=True)
                    return carry

                lax.fori_loop(0, nchunks // NBUF, outer, 0)
                for b in range(NBUF):
                    pltpu.make_async_copy(dummy, rows_v[b], ssem[b]).wait()
                plsc.subcore_barrier()

                def drain(kk, carry, step=step, g=g):
                    blk = kk * NS + sid

                    @pl.when(blk < n_blocks)
                    def _():
                        r0 = blk * 16
                        pltpu.sync_copy(acc.at[pl.ds(r0, 16)], dbuf)
                        for l in range(16):
                            s = _bcast(dinv2_v, r0 + l)
                            for f in range(FG // 16):
                                obuf[l, pl.ds(f * 16, 16)] = (
                                    dbuf[l, pl.ds(f * 16, 16)] * s)
                        pltpu.sync_copy(
                            obuf, us_hbm.at[step, g, pl.ds(r0, 16)])

                    return carry

                lax.fori_loop(0, nbt, drain, 0)
                plsc.subcore_barrier()

    return k


def _prep_u0(x, dinv):
    n, f = x.shape
    g1 = f // FG

    def body(x_ref, d_ref, o_ref):
        o_ref[0] = x_ref[...] * d_ref[...]

    return pl.pallas_call(
        body,
        grid=(g1, n // BN),
        in_specs=[
            pl.BlockSpec((BN, FG), lambda g, i: (i, g)),
            pl.BlockSpec((BN, 1), lambda g, i: (i, 0)),
        ],
        out_specs=pl.BlockSpec((1, BN, FG), lambda g, i: (g, i, 0)),
        out_shape=jax.ShapeDtypeStruct((g1, n, FG), jnp.float32),
    )(x, dinv)


def _dense(u0g, usg, w, b, scale, mrow, n_steps, out_grouped):
    n_groups, n, _ = u0g.shape
    hid = w.shape[0]
    fin = n_groups * FG
    g2 = hid // FG

    def body(u0_ref, us_ref, w_ref, b_ref, s_ref, m_ref, o_ref):
        z = jnp.zeros((BN, hid), jnp.float32)
        for g in range(n_groups):
            aug = us_ref[0, g]
            agg = m_ref[0, 0] * u0_ref[g]
            for i in range(1, n_steps + 1):
                agg = agg + m_ref[0, i] * us_ref[i - 1, g]
            wa = w_ref[:, g * FG:(g + 1) * FG]
            wb = w_ref[:, fin + g * FG:fin + (g + 1) * FG]
            z = z + lax.dot_general(aug, wa, (((1,), (1,)), ((), ())),
                                    preferred_element_type=jnp.float32)
            z = z + lax.dot_general(agg, wb, (((1,), (1,)), ((), ())),
                                    preferred_element_type=jnp.float32)
        if out_grouped:
            r = jax.nn.relu(z + s_ref[...] * b_ref[...])
            for g in range(g2):
                o_ref[g] = r[:, g * FG:(g + 1) * FG]
        else:
            r = jax.nn.relu(s_ref[...] * z + b_ref[...])
            o_ref[...] = r

    out_shape = (jax.ShapeDtypeStruct((g2, n, FG), jnp.float32)
                 if out_grouped else
                 jax.ShapeDtypeStruct((n, hid), jnp.float32))
    out_spec = (pl.BlockSpec((g2, BN, FG), lambda i: (0, i, 0))
                if out_grouped else
                pl.BlockSpec((BN, hid), lambda i: (i, 0)))
    return pl.pallas_call(
        body,
        grid=(n // BN,),
        in_specs=[
            pl.BlockSpec((n_groups, BN, FG), lambda i: (0, i, 0)),
            pl.BlockSpec((n_steps, n_groups, BN, FG),
                         lambda i: (0, 0, i, 0)),
            pl.BlockSpec((hid, 2 * fin), lambda i: (0, 0)),
            pl.BlockSpec((1, hid), lambda i: (0, 0)),
            pl.BlockSpec((BN, 1), lambda i: (i, 0)),
            pl.BlockSpec(memory_space=pltpu.SMEM),
        ],
        out_specs=out_spec,
        out_shape=out_shape,
    )(u0g, usg, w, b.reshape(1, hid), scale, mrow)


def _classify(h, w2, b2):
    n, hid = h.shape
    c = w2.shape[0]

    def body(h_ref, w_ref, b_ref, o_ref):
        logits = lax.dot_general(h_ref[...], w_ref[...],
                                 (((1,), (1,)), ((), ())),
                                 preferred_element_type=jnp.float32)
        logits = logits + b_ref[...]
        mx = jnp.max(logits, axis=1, keepdims=True)
        e = jnp.exp(logits - mx)
        lse = jnp.log(jnp.sum(e, axis=1, keepdims=True))
        o_ref[...] = logits - mx - lse

    return pl.pallas_call(
        body,
        grid=(n // BN,),
        in_specs=[
            pl.BlockSpec((BN, hid), lambda i: (i, 0)),
            pl.BlockSpec((c, hid), lambda i: (0, 0)),
            pl.BlockSpec((1, c), lambda i: (0, 0)),
        ],
        out_specs=pl.BlockSpec((BN, c), lambda i: (i, 0)),
        out_shape=jax.ShapeDtypeStruct((n, c), jnp.float32),
    )(h, w2, b2.reshape(1, c))


def kernel(x, edge_index, att, W0, b0, W1, b1, W2, b2):
    n, f_in = x.shape
    hid = W0.shape[0]
    ne = edge_index.shape[1]
    nlayer, kk1 = att.shape
    n_steps = kk1 - 1

    row, col = edge_index[0], edge_index[1]
    step_e = NC * NS * CH * NBUF
    ne_pad = -(-ne // step_e) * step_e
    nchunks_tile = ne_pad // (NC * NS * CH)
    rowp = jnp.pad(row, (0, ne_pad - ne))
    ne_t = nchunks_tile * CH
    padpos = jnp.arange(ne, ne_pad, dtype=jnp.int32)
    trash = (n + (padpos // ne_t) * TRASH_PER_TILE
             + (padpos % TRASH_PER_TILE))
    colp = jnp.concatenate([col, trash])

    racc = n + NC * NS * TRASH_PER_TILE
    zeros_deg = jnp.zeros((racc, DW), jnp.float32)
    ones_deg = jnp.ones((CH, DW), jnp.float32)
    zeros_fg = jnp.zeros((racc, FG), jnp.float32)

    deg2 = _make_deg_kernel(n, ne_pad)(colp, ones_deg, zeros_deg)
    deg = deg2[0, :, 0] + deg2[1, :, 0]
    dinv2 = (1.0 / deg).astype(jnp.float32)
    dinv = jnp.sqrt(dinv2)
    dsqrt = deg * dinv
    dinv_c = dinv[:, None]
    dsqrt_c = dsqrt[:, None]

    mpad = jnp.pad(att, ((0, 0), (0, 8 - kk1))).astype(jnp.float32)

    u0 = _prep_u0(x, dinv_c)
    us1 = _make_layer_kernel(n, ne_pad, f_in // FG, n_steps)(
        u0, rowp, colp, dinv2, zeros_fg)
    u0p = _dense(u0, us1, W0, b0, dinv_c, mpad[0:1], n_steps,
                 out_grouped=True)

    us2 = _make_layer_kernel(n, ne_pad, hid // FG, n_steps)(
        u0p, rowp, colp, dinv2, zeros_fg)
    h2 = _dense(u0p, us2, W1, b1, dsqrt_c, mpad[1:2], n_steps,
                out_grouped=False)

    return _classify(h2, W2, b2)


# speedup vs baseline: 1.0631x; 1.0631x over previous
import functools

import jax
import jax.numpy as jnp
from jax import lax
from jax.experimental import pallas as pl
from jax.experimental.pallas import tpu as pltpu
from jax.experimental.pallas import tpu_sc as plsc

NC = 2
NS = 16
CH = 128
IB = 8
FG = 128
DW = 16
BN = 1000


def _mesh():
    return plsc.VectorSubcoreMesh(core_axis_name="c", subcore_axis_name="s",
                                  num_cores=NC, num_subcores=NS)


def _bcast(ref, i):
    return plsc.load_gather(ref, [jnp.full((16,), i, jnp.int32)])


TRASH_PER_TILE = 8
NBUF = 1


def _make_deg_kernel(n_nodes, ne_pad):
    nchunks = ne_pad // (NC * NS * CH)
    racc = n_nodes + NC * NS * TRASH_PER_TILE

    @functools.partial(
        pl.kernel,
        out_type=jax.ShapeDtypeStruct((NC, n_nodes, DW), jnp.float32),
        mesh=_mesh(),
        compiler_params=pltpu.CompilerParams(needs_layout_passes=False),
        scratch_types=[
            pltpu.VMEM((CH,), jnp.int32),
            pltpu.VMEM((CH, DW), jnp.float32),
            pltpu.VMEM_SHARED((racc, DW), jnp.float32),
        ],
    )
    def k(colp_hbm, ones_hbm, zeros_hbm, out_hbm, dst_v, ones_v, acc):
        cid = lax.axis_index("c")
        sid = lax.axis_index("s")
        wid = sid * NC + cid
        pltpu.sync_copy(ones_hbm, ones_v)

        @pl.when(sid == 0)
        def _():
            pltpu.sync_copy(zeros_hbm, acc)

        plsc.subcore_barrier()

        def chunk(kk, carry):
            e0 = (wid * nchunks + kk) * CH
            pltpu.sync_copy(colp_hbm.at[pl.ds(e0, CH)], dst_v)
            pltpu.sync_copy(ones_v, acc.at[dst_v], add=True)
            return carry

        lax.fori_loop(0, nchunks, chunk, 0)
        plsc.subcore_barrier()

        @pl.when(sid == 0)
        def _():
            pltpu.sync_copy(acc.at[pl.ds(0, n_nodes)], out_hbm.at[cid])

        plsc.subcore_barrier()

    return k


def _make_layer_kernel(n_nodes, ne_pad, n_groups, n_steps):
    nchunks = ne_pad // (NC * NS * CH)
    racc = n_nodes + NC * NS * TRASH_PER_TILE
    gpc = n_groups // NC
    n_blocks = n_nodes // 16
    nbt = -(-n_blocks // NS)

    @functools.partial(
        pl.kernel,
        out_type=jax.ShapeDtypeStruct((n_steps, n_groups, n_nodes, FG),
                                      jnp.float32),
        mesh=_mesh(),
        compiler_params=pltpu.CompilerParams(needs_layout_passes=False),
        scratch_types=[
            [pltpu.VMEM((CH,), jnp.int32) for _ in range(NBUF)],
            [pltpu.VMEM((CH,), jnp.int32) for _ in range(NBUF)],
            [pltpu.VMEM((CH, FG), jnp.float32) for _ in range(NBUF)],
            [pltpu.SemaphoreType.DMA for _ in range(NBUF)],
            [pltpu.SemaphoreType.DMA for _ in range(NBUF)],
            pltpu.VMEM((16, FG), jnp.float32),
            pltpu.VMEM((16, FG), jnp.float32),
            pltpu.VMEM((n_nodes,), jnp.float32),
            pltpu.VMEM_SHARED((racc, FG), jnp.float32),
        ],
    )
    def k(u0_hbm, rowp_hbm, colp_hbm, dinv2_hbm, zeros_hbm, us_hbm,
          idx_v, dst_v, rows_v, gsem, ssem, dbuf, obuf, dinv2_v, acc):
        cid = lax.axis_index("c")
        sid = lax.axis_index("s")
        wid = sid * NC + cid
        pltpu.sync_copy(dinv2_hbm, dinv2_v)
        dummy = u0_hbm.at[0, pl.ds(0, CH)]

        for step in range(n_steps):
            for gg in range(gpc):
                g = cid * gpc + gg

                @pl.when(sid == 0)
                def _():
                    pltpu.sync_copy(zeros_hbm, acc)

                plsc.subcore_barrier()

                if step == 0:
                    gsrc = u0_hbm.at[g]
                else:
                    gsrc = us_hbm.at[step - 1, g]

                def chunk(kk, carry, gsrc=gsrc):
                    e0 = (wid * nchunks + kk) * CH
                    pltpu.sync_copy(rowp_hbm.at[pl.ds(e0, CH)], idx_v[0])
                    pltpu.sync_copy(colp_hbm.at[pl.ds(e0, CH)], dst_v[0])
                    pltpu.async_copy(gsrc.at[idx_v[0]], rows_v[0],
                                     gsem[0]).wait()
                    pltpu.sync_copy(rows_v[0], acc.at[dst_v[0]], add=True)
                    return carry

                lax.fori_loop(0, nchunks, chunk, 0)
                plsc.subcore_barrier()

                def drain(kk, carry, step=step, g=g):
                    blk = kk * NS + sid

                    @pl.when(blk < n_blocks)
                    def _():
                        r0 = blk * 16
                        pltpu.sync_copy(acc.at[pl.ds(r0, 16)], dbuf)
                        for l in range(16):
                            s = _bcast(dinv2_v, r0 + l)
                            for f in range(FG // 16):
                                obuf[l, pl.ds(f * 16, 16)] = (
                                    dbuf[l, pl.ds(f * 16, 16)] * s)
                        pltpu.sync_copy(
                            obuf, us_hbm.at[step, g, pl.ds(r0, 16)])

                    return carry

                lax.fori_loop(0, nbt, drain, 0)
                plsc.subcore_barrier()

    return k


def _prep_u0(x, dinv):
    n, f = x.shape
    g1 = f // FG

    def body(x_ref, d_ref, o_ref):
        o_ref[0] = x_ref[...] * d_ref[...]

    return pl.pallas_call(
        body,
        grid=(g1, n // BN),
        in_specs=[
            pl.BlockSpec((BN, FG), lambda g, i: (i, g)),
            pl.BlockSpec((BN, 1), lambda g, i: (i, 0)),
        ],
        out_specs=pl.BlockSpec((1, BN, FG), lambda g, i: (g, i, 0)),
        out_shape=jax.ShapeDtypeStruct((g1, n, FG), jnp.float32),
    )(x, dinv)


def _dense(u0g, usg, w, b, scale, mrow, n_steps, out_grouped):
    n_groups, n, _ = u0g.shape
    hid = w.shape[0]
    fin = n_groups * FG
    g2 = hid // FG

    def body(u0_ref, us_ref, w_ref, b_ref, s_ref, m_ref, o_ref):
        z = jnp.zeros((BN, hid), jnp.float32)
        for g in range(n_groups):
            aug = us_ref[0, g]
            agg = m_ref[0, 0] * u0_ref[g]
            for i in range(1, n_steps + 1):
                agg = agg + m_ref[0, i] * us_ref[i - 1, g]
            wa = w_ref[:, g * FG:(g + 1) * FG]
            wb = w_ref[:, fin + g * FG:fin + (g + 1) * FG]
            z = z + lax.dot_general(aug, wa, (((1,), (1,)), ((), ())),
                                    preferred_element_type=jnp.float32)
            z = z + lax.dot_general(agg, wb, (((1,), (1,)), ((), ())),
                                    preferred_element_type=jnp.float32)
        if out_grouped:
            r = jax.nn.relu(z + s_ref[...] * b_ref[...])
            for g in range(g2):
                o_ref[g] = r[:, g * FG:(g + 1) * FG]
        else:
            r = jax.nn.relu(s_ref[...] * z + b_ref[...])
            o_ref[...] = r

    out_shape = (jax.ShapeDtypeStruct((g2, n, FG), jnp.float32)
                 if out_grouped else
                 jax.ShapeDtypeStruct((n, hid), jnp.float32))
    out_spec = (pl.BlockSpec((g2, BN, FG), lambda i: (0, i, 0))
                if out_grouped else
                pl.BlockSpec((BN, hid), lambda i: (i, 0)))
    return pl.pallas_call(
        body,
        grid=(n // BN,),
        in_specs=[
            pl.BlockSpec((n_groups, BN, FG), lambda i: (0, i, 0)),
            pl.BlockSpec((n_steps, n_groups, BN, FG),
                         lambda i: (0, 0, i, 0)),
            pl.BlockSpec((hid, 2 * fin), lambda i: (0, 0)),
            pl.BlockSpec((1, hid), lambda i: (0, 0)),
            pl.BlockSpec((BN, 1), lambda i: (i, 0)),
            pl.BlockSpec(memory_space=pltpu.SMEM),
        ],
        out_specs=out_spec,
        out_shape=out_shape,
    )(u0g, usg, w, b.reshape(1, hid), scale, mrow)


def _classify(h, w2, b2):
    n, hid = h.shape
    c = w2.shape[0]

    def body(h_ref, w_ref, b_ref, o_ref):
        logits = lax.dot_general(h_ref[...], w_ref[...],
                                 (((1,), (1,)), ((), ())),
                                 preferred_element_type=jnp.float32)
        logits = logits + b_ref[...]
        mx = jnp.max(logits, axis=1, keepdims=True)
        e = jnp.exp(logits - mx)
        lse = jnp.log(jnp.sum(e, axis=1, keepdims=True))
        o_ref[...] = logits - mx - lse

    return pl.pallas_call(
        body,
        grid=(n // BN,),
        in_specs=[
            pl.BlockSpec((BN, hid), lambda i: (i, 0)),
            pl.BlockSpec((c, hid), lambda i: (0, 0)),
            pl.BlockSpec((1, c), lambda i: (0, 0)),
        ],
        out_specs=pl.BlockSpec((BN, c), lambda i: (i, 0)),
        out_shape=jax.ShapeDtypeStruct((n, c), jnp.float32),
    )(h, w2, b2.reshape(1, c))


def kernel(x, edge_index, att, W0, b0, W1, b1, W2, b2):
    n, f_in = x.shape
    hid = W0.shape[0]
    ne = edge_index.shape[1]
    nlayer, kk1 = att.shape
    n_steps = kk1 - 1

    row, col = edge_index[0], edge_index[1]
    step_e = NC * NS * CH
    ne_pad = -(-ne // step_e) * step_e
    nchunks_tile = ne_pad // (NC * NS * CH)
    rowp = jnp.pad(row, (0, ne_pad - ne))
    ne_t = nchunks_tile * CH
    padpos = jnp.arange(ne, ne_pad, dtype=jnp.int32)
    trash = (n + (padpos // ne_t) * TRASH_PER_TILE
             + (padpos % TRASH_PER_TILE))
    colp = jnp.concatenate([col, trash])

    racc = n + NC * NS * TRASH_PER_TILE
    zeros_deg = jnp.zeros((racc, DW), jnp.float32)
    ones_deg = jnp.ones((CH, DW), jnp.float32)
    zeros_fg = jnp.zeros((racc, FG), jnp.float32)

    deg2 = _make_deg_kernel(n, ne_pad)(colp, ones_deg, zeros_deg)
    deg = deg2[0, :, 0] + deg2[1, :, 0]
    dinv2 = (1.0 / deg).astype(jnp.float32)
    dinv = jnp.sqrt(dinv2)
    dsqrt = deg * dinv
    dinv_c = dinv[:, None]
    dsqrt_c = dsqrt[:, None]

    mpad = jnp.pad(att, ((0, 0), (0, 8 - kk1))).astype(jnp.float32)

    u0 = _prep_u0(x, dinv_c)
    us1 = _make_layer_kernel(n, ne_pad, f_in // FG, n_steps)(
        u0, rowp, colp, dinv2, zeros_fg)
    u0p = _dense(u0, us1, W0, b0, dinv_c, mpad[0:1], n_steps,
                 out_grouped=True)

    us2 = _make_layer_kernel(n, ne_pad, hid // FG, n_steps)(
        u0p, rowp, colp, dinv2, zeros_fg)
    h2 = _dense(u0p, us2, W1, b1, dsqrt_c, mpad[1:2], n_steps,
                out_grouped=False)

    return _classify(h2, W2, b2)
